# baseline (speedup 1.0000x reference)
import functools

import jax
import jax.numpy as jnp
from jax import lax
from jax.experimental import pallas as pl
from jax.experimental.pallas import tpu as pltpu
from jax.experimental.pallas import tpu_sc as plsc

_L = 16
_RBLK = 8
_SC_RAYS = 1024


def _ray_sampler(cam2world_matrix, intrinsics, resolution):
    N = cam2world_matrix.shape[0]
    M = resolution * resolution
    cam_locs_world = cam2world_matrix[:, :3, 3]
    fx = intrinsics[:, 0, 0]
    fy = intrinsics[:, 1, 1]
    cx = intrinsics[:, 0, 2]
    cy = intrinsics[:, 1, 2]
    sk = intrinsics[:, 0, 1]
    ii, jj = jnp.meshgrid(jnp.arange(resolution, dtype=jnp.float32),
                          jnp.arange(resolution, dtype=jnp.float32), indexing='ij')
    uv = jnp.stack([ii, jj]) * (1.0 / resolution) + (0.5 / resolution)
    uv = jnp.flip(uv, axis=0).reshape(2, -1).T
    uv = jnp.broadcast_to(uv[None], (N, M, 2))
    x_cam = uv[:, :, 0]
    y_cam = uv[:, :, 1]
    z_cam = jnp.ones((N, M), dtype=jnp.float32)
    x_lift = (x_cam - cx[:, None] + cy[:, None] * sk[:, None] / fy[:, None]
              - sk[:, None] * y_cam / fy[:, None]) / fx[:, None] * z_cam
    y_lift = (y_cam - cy[:, None]) / fy[:, None] * z_cam
    cam_rel_points = jnp.stack(
        [x_lift, y_lift, z_cam, jnp.ones((N, M), dtype=jnp.float32)], axis=-1)
    world_rel_points = jnp.einsum('nij,nmj->nmi', cam2world_matrix, cam_rel_points)[:, :, :3]
    ray_dirs = world_rel_points - cam_locs_world[:, None, :]
    ray_dirs = ray_dirs / jnp.linalg.norm(ray_dirs, axis=2, keepdims=True)
    ray_origins = jnp.broadcast_to(cam_locs_world[:, None, :], ray_dirs.shape)
    return ray_origins, ray_dirs


def _nn_sqdist_body(n_batches, n_total_rays, n_rays, n_pts, pred_hbm, nrm_hbm,
                    pc_hbm, sp_hbm, out_hbm,
                    pcx, pcy, pcz, sp, prx, pry, prz, prn, outv):
    cid = lax.axis_index("c")
    sid = lax.axis_index("s")
    wid = sid * 2 + cid
    wpb = 32 // n_batches
    b = wid // wpb
    rbase0 = (wid % wpb) * n_rays

    pc0 = b * (3 * n_pts)
    pr0 = b * (3 * n_total_rays) + rbase0
    pltpu.sync_copy(pc_hbm.at[pl.ds(pc0, n_pts)], pcx)
    pltpu.sync_copy(pc_hbm.at[pl.ds(pc0 + n_pts, n_pts)], pcy)
    pltpu.sync_copy(pc_hbm.at[pl.ds(pc0 + 2 * n_pts, n_pts)], pcz)
    pltpu.sync_copy(sp_hbm.at[pl.ds(b * n_pts, n_pts)], sp)
    pltpu.sync_copy(pred_hbm.at[pl.ds(pr0, n_rays)], prx)
    pltpu.sync_copy(pred_hbm.at[pl.ds(pr0 + n_total_rays, n_rays)], pry)
    pltpu.sync_copy(pred_hbm.at[pl.ds(pr0 + 2 * n_total_rays, n_rays)], prz)
    pltpu.sync_copy(nrm_hbm.at[pl.ds(b * n_total_rays + rbase0, n_rays)], prn)

    n_chunks = n_pts // _L

    inf = jnp.full((_L,), jnp.inf, dtype=jnp.float32)
    lanes = lax.iota(jnp.int32, _L)

    dnums = lax.GatherDimensionNumbers(
        offset_dims=(), collapsed_slice_dims=(0,), start_index_map=(0,))

    def _permute(v, idx):
        return lax.gather(v, idx.reshape(_L, 1), dnums, slice_sizes=(1,),
                          mode=lax.GatherScatterMode.PROMISE_IN_BOUNDS)

    def _splat(v, i):
        return _permute(v, jnp.full((_L,), i, dtype=jnp.int32))

    def ray_block(rb, carry):
        rbase = rb * _L
        vx = prx[pl.ds(rbase, _L)]
        vy = pry[pl.ds(rbase, _L)]
        vz = prz[pl.ds(rbase, _L)]
        vn = prn[pl.ds(rbase, _L)]
        outvec = jnp.zeros((_L,), dtype=jnp.float32)
        for half in range(_L // _RBLK):
            ax, ay, az, nrm = [], [], [], []
            for i in range(_RBLK):
                ax.append(_splat(vx, half * _RBLK + i) * -2.0)
                ay.append(_splat(vy, half * _RBLK + i) * -2.0)
                az.append(_splat(vz, half * _RBLK + i) * -2.0)
                nrm.append(_splat(vn, half * _RBLK + i))

            @plsc.parallel_loop(0, n_chunks, step=1, unroll=4,
                                carry=(inf,) * _RBLK)
            def accs(j, accs_in):
                base = j * _L
                x = pcx[pl.ds(base, _L)]
                y = pcy[pl.ds(base, _L)]
                z = pcz[pl.ds(base, _L)]
                s0 = sp[pl.ds(base, _L)]
                out = []
                for i in range(_RBLK):
                    t = s0 + x * ax[i]
                    t = t + y * ay[i]
                    t = t + z * az[i]
                    out.append(jnp.minimum(accs_in[i], t))
                return tuple(out)

            for i in range(_RBLK):
                v = accs[i] + nrm[i]
                for sh in (1, 2, 4, 8):
                    v = jnp.minimum(v, _permute(v, lanes ^ sh))
                outvec = jnp.where(lanes == half * _RBLK + i, v, outvec)
        outv[pl.ds(rbase, _L)] = outvec
        return carry

    lax.fori_loop(0, n_rays // _L, ray_block, 0)
    pltpu.sync_copy(outv, out_hbm.at[pl.ds(b * n_total_rays + rbase0, n_rays)])


def _tc_body(p_ref, n_ref, q_ref, s_ref, o_ref):
    p = p_ref[0]
    q = q_ref[0]
    dot = jnp.dot(p, q, preferred_element_type=jnp.float32)
    m = jnp.min(dot + s_ref[0, 0][None, :], axis=1)
    o_ref[0, 0] = m + n_ref[0, 0]


def _nn_tc(pred_rows, nrm_tc, pc_bf, sp, TR=512):
    B, MT, _ = pred_rows.shape
    K = pc_bf.shape[2]
    return pl.pallas_call(
        _tc_body,
        grid=(B, MT // TR),
        in_specs=[
            pl.BlockSpec((1, TR, 3), lambda b, r: (b, r, 0)),
            pl.BlockSpec((1, 1, TR), lambda b, r: (b, 0, r)),
            pl.BlockSpec((1, 3, K), lambda b, r: (b, 0, 0)),
            pl.BlockSpec((1, 1, K), lambda b, r: (b, 0, 0)),
        ],
        out_specs=pl.BlockSpec((1, 1, TR), lambda b, r: (b, 0, r)),
        out_shape=jax.ShapeDtypeStruct((B, 1, MT), jnp.float32),
    )(pred_rows * -2.0, nrm_tc.reshape(B, 1, MT), pc_bf,
      sp.reshape(B, 1, K)).reshape(B, MT)


@functools.partial(jax.jit, static_argnums=())
def _nn_sqdist(predT, pcT):
    B, _, M = predT.shape
    K = pcT.shape[2]
    sp = jnp.sum(pcT * pcT, axis=1)
    nrm = jnp.sum(predT * predT, axis=1)

    def _bf16_round(t):
        i = lax.bitcast_convert_type(t, jnp.int32)
        r = i + jnp.int32(0x7FFF) + ((i >> 16) & 1)
        return lax.bitcast_convert_type(r & jnp.int32(-65536), jnp.float32)

    pc_bf = _bf16_round(pcT)
    pred_bf = _bf16_round(predT)

    S = _SC_RAYS
    n_rays = (B * S) // 32
    body = functools.partial(_nn_sqdist_body, B, S, n_rays, K)
    kfn = pl.kernel(
        body,
        out_type=jax.ShapeDtypeStruct((B * S,), jnp.float32),
        mesh=plsc.VectorSubcoreMesh(core_axis_name="c", subcore_axis_name="s",
                                    num_cores=2, num_subcores=16),
        scratch_types=[
            pltpu.VMEM((K,), jnp.float32),
            pltpu.VMEM((K,), jnp.float32),
            pltpu.VMEM((K,), jnp.float32),
            pltpu.VMEM((K,), jnp.float32),
            pltpu.VMEM((n_rays,), jnp.float32),
            pltpu.VMEM((n_rays,), jnp.float32),
            pltpu.VMEM((n_rays,), jnp.float32),
            pltpu.VMEM((n_rays,), jnp.float32),
            pltpu.VMEM((n_rays,), jnp.float32),
        ],
    )
    d1_sc = kfn(pred_bf[:, :, :S].reshape(-1), nrm[:, :S].reshape(-1),
                pc_bf.reshape(-1), sp.reshape(-1)).reshape(B, S)
    pred_rows = jnp.transpose(pred_bf[:, :, S:], (0, 2, 1))
    d1_tc = _nn_tc(pred_rows, nrm[:, S:], pc_bf, sp)
    return jnp.concatenate([d1_sc, d1_tc], axis=1)


def kernel(c, image, image_depth, pc, neural_rendering_resolution):
    B = c.shape[0]
    res = image.shape[-1]
    pc3 = pc[..., :3]
    depth = image_depth.reshape(B, -1)
    cam2world_matrix = c[:, :16].reshape(-1, 4, 4)
    intrinsics = c[:, 16:25].reshape(-1, 3, 3)
    ray_origins, ray_dirs = _ray_sampler(cam2world_matrix, intrinsics, res)
    origin = ray_origins[:, 0]
    dist = jnp.sqrt(jnp.sum((origin[:, None, :] - pc3) ** 2, axis=2))
    max_distance = jnp.max(dist, axis=1)
    pred_pos = depth[..., None] * ray_dirs + ray_origins
    predT = jnp.transpose(pred_pos, (0, 2, 1))
    pcT = jnp.transpose(pc3, (0, 2, 1))
    d1 = _nn_sqdist(predT, pcT)
    mask = depth < max_distance[:, None]
    masked = jnp.where(mask, d1, 0.0)
    count = jnp.sum(mask, axis=1)
    loss = jnp.sum(masked, axis=1) / count
    out = loss.reshape(B, 1).astype(jnp.float32)
    return out + (jnp.asarray(neural_rendering_resolution) * 0).astype(jnp.float32)


# speedup vs baseline: 1.0969x; 1.0969x over previous
import functools

import jax
import jax.numpy as jnp
from jax import lax
from jax.experimental import pallas as pl
from jax.experimental.pallas import tpu as pltpu
from jax.experimental.pallas import tpu_sc as plsc

_L = 16
_RBLK = 8
_SC_RAYS = 768


def _ray_sampler(cam2world_matrix, intrinsics, resolution):
    N = cam2world_matrix.shape[0]
    M = resolution * resolution
    cam_locs_world = cam2world_matrix[:, :3, 3]
    fx = intrinsics[:, 0, 0]
    fy = intrinsics[:, 1, 1]
    cx = intrinsics[:, 0, 2]
    cy = intrinsics[:, 1, 2]
    sk = intrinsics[:, 0, 1]
    ii, jj = jnp.meshgrid(jnp.arange(resolution, dtype=jnp.float32),
                          jnp.arange(resolution, dtype=jnp.float32), indexing='ij')
    uv = jnp.stack([ii, jj]) * (1.0 / resolution) + (0.5 / resolution)
    uv = jnp.flip(uv, axis=0).reshape(2, -1).T
    uv = jnp.broadcast_to(uv[None], (N, M, 2))
    x_cam = uv[:, :, 0]
    y_cam = uv[:, :, 1]
    z_cam = jnp.ones((N, M), dtype=jnp.float32)
    x_lift = (x_cam - cx[:, None] + cy[:, None] * sk[:, None] / fy[:, None]
              - sk[:, None] * y_cam / fy[:, None]) / fx[:, None] * z_cam
    y_lift = (y_cam - cy[:, None]) / fy[:, None] * z_cam
    cam_rel_points = jnp.stack(
        [x_lift, y_lift, z_cam, jnp.ones((N, M), dtype=jnp.float32)], axis=-1)
    world_rel_points = jnp.einsum('nij,nmj->nmi', cam2world_matrix, cam_rel_points)[:, :, :3]
    ray_dirs = world_rel_points - cam_locs_world[:, None, :]
    ray_dirs = ray_dirs / jnp.linalg.norm(ray_dirs, axis=2, keepdims=True)
    ray_origins = jnp.broadcast_to(cam_locs_world[:, None, :], ray_dirs.shape)
    return ray_origins, ray_dirs


def _nn_sqdist_body(n_batches, n_total_rays, n_rays, n_pts, pred_hbm, nrm_hbm,
                    pc_hbm, sp_hbm, out_hbm,
                    pcx, pcy, pcz, sp, prx, pry, prz, prn, outv):
    cid = lax.axis_index("c")
    sid = lax.axis_index("s")
    wid = sid * 2 + cid
    wpb = 32 // n_batches
    b = wid // wpb
    rbase0 = (wid % wpb) * n_rays

    pc0 = b * (3 * n_pts)
    pr0 = b * (3 * n_total_rays) + rbase0
    pltpu.sync_copy(pc_hbm.at[pl.ds(pc0, n_pts)], pcx)
    pltpu.sync_copy(pc_hbm.at[pl.ds(pc0 + n_pts, n_pts)], pcy)
    pltpu.sync_copy(pc_hbm.at[pl.ds(pc0 + 2 * n_pts, n_pts)], pcz)
    pltpu.sync_copy(sp_hbm.at[pl.ds(b * n_pts, n_pts)], sp)
    pltpu.sync_copy(pred_hbm.at[pl.ds(pr0, n_rays)], prx)
    pltpu.sync_copy(pred_hbm.at[pl.ds(pr0 + n_total_rays, n_rays)], pry)
    pltpu.sync_copy(pred_hbm.at[pl.ds(pr0 + 2 * n_total_rays, n_rays)], prz)
    pltpu.sync_copy(nrm_hbm.at[pl.ds(b * n_total_rays + rbase0, n_rays)], prn)

    n_chunks = n_pts // _L

    inf = jnp.full((_L,), jnp.inf, dtype=jnp.float32)
    lanes = lax.iota(jnp.int32, _L)

    dnums = lax.GatherDimensionNumbers(
        offset_dims=(), collapsed_slice_dims=(0,), start_index_map=(0,))

    def _permute(v, idx):
        return lax.gather(v, idx.reshape(_L, 1), dnums, slice_sizes=(1,),
                          mode=lax.GatherScatterMode.PROMISE_IN_BOUNDS)

    def _splat(v, i):
        return _permute(v, jnp.full((_L,), i, dtype=jnp.int32))

    def ray_block(rb, carry):
        rbase = rb * _L
        vx = prx[pl.ds(rbase, _L)]
        vy = pry[pl.ds(rbase, _L)]
        vz = prz[pl.ds(rbase, _L)]
        vn = prn[pl.ds(rbase, _L)]
        outvec = jnp.zeros((_L,), dtype=jnp.float32)
        for half in range(_L // _RBLK):
            ax, ay, az, nrm = [], [], [], []
            for i in range(_RBLK):
                ax.append(_splat(vx, half * _RBLK + i) * -2.0)
                ay.append(_splat(vy, half * _RBLK + i) * -2.0)
                az.append(_splat(vz, half * _RBLK + i) * -2.0)
                nrm.append(_splat(vn, half * _RBLK + i))

            @plsc.parallel_loop(0, n_chunks, step=1, unroll=4,
                                carry=(inf,) * _RBLK)
            def accs(j, accs_in):
                base = j * _L
                x = pcx[pl.ds(base, _L)]
                y = pcy[pl.ds(base, _L)]
                z = pcz[pl.ds(base, _L)]
                s0 = sp[pl.ds(base, _L)]
                out = []
                for i in range(_RBLK):
                    t = s0 + x * ax[i]
                    t = t + y * ay[i]
                    t = t + z * az[i]
                    out.append(jnp.minimum(accs_in[i], t))
                return tuple(out)

            for i in range(_RBLK):
                v = accs[i] + nrm[i]
                for sh in (1, 2, 4, 8):
                    v = jnp.minimum(v, _permute(v, lanes ^ sh))
                outvec = jnp.where(lanes == half * _RBLK + i, v, outvec)
        outv[pl.ds(rbase, _L)] = outvec
        return carry

    lax.fori_loop(0, n_rays // _L, ray_block, 0)
    pltpu.sync_copy(outv, out_hbm.at[pl.ds(b * n_total_rays + rbase0, n_rays)])


def _tc_body(p_ref, n_ref, q_ref, s_ref, o_ref):
    p = p_ref[0]
    q = q_ref[0]
    dot = jnp.dot(p, q, preferred_element_type=jnp.float32)
    m = jnp.min(dot + s_ref[0, 0][None, :], axis=1)
    o_ref[0, 0] = m + n_ref[0, 0]


def _nn_tc(pred_rows, nrm_tc, pc_bf, sp, TR=256):
    B, MT, _ = pred_rows.shape
    K = pc_bf.shape[2]
    return pl.pallas_call(
        _tc_body,
        grid=(B, MT // TR),
        in_specs=[
            pl.BlockSpec((1, TR, 3), lambda b, r: (b, r, 0)),
            pl.BlockSpec((1, 1, TR), lambda b, r: (b, 0, r)),
            pl.BlockSpec((1, 3, K), lambda b, r: (b, 0, 0)),
            pl.BlockSpec((1, 1, K), lambda b, r: (b, 0, 0)),
        ],
        out_specs=pl.BlockSpec((1, 1, TR), lambda b, r: (b, 0, r)),
        out_shape=jax.ShapeDtypeStruct((B, 1, MT), jnp.float32),
    )(pred_rows * -2.0, nrm_tc.reshape(B, 1, MT), pc_bf,
      sp.reshape(B, 1, K)).reshape(B, MT)


@functools.partial(jax.jit, static_argnums=())
def _nn_sqdist(predT, pcT):
    B, _, M = predT.shape
    K = pcT.shape[2]
    sp = jnp.sum(pcT * pcT, axis=1)
    nrm = jnp.sum(predT * predT, axis=1)

    def _bf16_round(t):
        i = lax.bitcast_convert_type(t, jnp.int32)
        r = i + jnp.int32(0x7FFF) + ((i >> 16) & 1)
        return lax.bitcast_convert_type(r & jnp.int32(-65536), jnp.float32)

    pc_bf = _bf16_round(pcT)
    pred_bf = _bf16_round(predT)

    S = _SC_RAYS
    n_rays = (B * S) // 32
    body = functools.partial(_nn_sqdist_body, B, S, n_rays, K)
    kfn = pl.kernel(
        body,
        out_type=jax.ShapeDtypeStruct((B * S,), jnp.float32),
        mesh=plsc.VectorSubcoreMesh(core_axis_name="c", subcore_axis_name="s",
                                    num_cores=2, num_subcores=16),
        scratch_types=[
            pltpu.VMEM((K,), jnp.float32),
            pltpu.VMEM((K,), jnp.float32),
            pltpu.VMEM((K,), jnp.float32),
            pltpu.VMEM((K,), jnp.float32),
            pltpu.VMEM((n_rays,), jnp.float32),
            pltpu.VMEM((n_rays,), jnp.float32),
            pltpu.VMEM((n_rays,), jnp.float32),
            pltpu.VMEM((n_rays,), jnp.float32),
            pltpu.VMEM((n_rays,), jnp.float32),
        ],
    )
    d1_sc = kfn(pred_bf[:, :, :S].reshape(-1), nrm[:, :S].reshape(-1),
                pc_bf.reshape(-1), sp.reshape(-1)).reshape(B, S)
    pred_rows = jnp.transpose(pred_bf[:, :, S:], (0, 2, 1))
    d1_tc = _nn_tc(pred_rows, nrm[:, S:], pc_bf, sp)
    return jnp.concatenate([d1_sc, d1_tc], axis=1)


def kernel(c, image, image_depth, pc, neural_rendering_resolution):
    B = c.shape[0]
    res = image.shape[-1]
    pc3 = pc[..., :3]
    depth = image_depth.reshape(B, -1)
    cam2world_matrix = c[:, :16].reshape(-1, 4, 4)
    intrinsics = c[:, 16:25].reshape(-1, 3, 3)
    ray_origins, ray_dirs = _ray_sampler(cam2world_matrix, intrinsics, res)
    origin = ray_origins[:, 0]
    dist = jnp.sqrt(jnp.sum((origin[:, None, :] - pc3) ** 2, axis=2))
    max_distance = jnp.max(dist, axis=1)
    pred_pos = depth[..., None] * ray_dirs + ray_origins
    predT = jnp.transpose(pred_pos, (0, 2, 1))
    pcT = jnp.transpose(pc3, (0, 2, 1))
    d1 = _nn_sqdist(predT, pcT)
    mask = depth < max_distance[:, None]
    masked = jnp.where(mask, d1, 0.0)
    count = jnp.sum(mask, axis=1)
    loss = jnp.sum(masked, axis=1) / count
    out = loss.reshape(B, 1).astype(jnp.float32)
    return out + (jnp.asarray(neural_rendering_resolution) * 0).astype(jnp.float32)
